# baseline (speedup 1.0000x reference)
import functools

import jax
import jax.numpy as jnp
from jax import lax
from jax.experimental import pallas as pl
from jax.experimental.pallas import tpu as pltpu
from jax.experimental.pallas import tpu_sc as plsc

N_NODES = 10000
N_EDGES = 320000
D = 128

NP = 10240
DH = D // 2
N_TILES = 16
NODES_PER_TILE = NP // N_TILES
EDGES_PER_TILE = N_EDGES // N_TILES
CHUNK = 100
CHUNKS_PER_TILE = EDGES_PER_TILE // CHUNK
NBUF = 5
N_BATCHES = CHUNKS_PER_TILE // NBUF
ROW_BLK = 32
BLKS_PER_TILE = NODES_PER_TILE // ROW_BLK
NGRP = NODES_PER_TILE // 16
MERGE = 128


def _rsqrt16(d):
    i = plsc.bitcast(d, jnp.int32)
    i = jnp.int32(0x5F3759DF) - (i >> 1)
    y = plsc.bitcast(i, jnp.float32)
    for _ in range(3):
        y = y * (1.5 - 0.5 * d * y * y)
    return jnp.where(d > 0.5, y, 0.0)


def _scale_blk(xb_v, p, dis_v, base):
    @pl.loop(0, ROW_BLK // 16)
    def _(j):
        dv = dis_v[base // 16 + j]
        for rr in range(16):
            dsc = dv[rr]
            for f in range(DH // 16):
                xb_v[p, j * 16 + rr, pl.ds(f * 16, 16)] = (
                    xb_v[p, j * 16 + rr, pl.ds(f * 16, 16)] * dsc)


def _gcn_body(xh, row2_h, col_h, out_h, y_h,
              row2_v, col_v, degp_v, dis_v, idx_v, bufs_v, xb_v,
              sem_g, sem_s, sem_i, sem_o,
              sh_deg, sh_out):
    c = lax.axis_index("c")
    s = lax.axis_index("s")
    nb = s * NODES_PER_TILE

    zeros16 = jnp.zeros((16,), jnp.float32)
    ones16 = jnp.ones((16,), jnp.float32)
    iota16 = lax.iota(jnp.int32, 16)

    pltpu.sync_copy(row2_h.at[s], row2_v)
    pltpu.sync_copy(col_h.at[s], col_v)

    @pl.loop(0, NP // 16)
    def _(i):
        degp_v[i] = zeros16

    @pl.loop(0, CHUNKS_PER_TILE)
    def _(g):
        for k in range(CHUNK // 16):
            idx = row2_v[g, pl.ds(k * 16, 16)]
            plsc.addupdate_scatter(degp_v, [idx >> 4, idx & 15], ones16)

    @pl.loop(0, NGRP)
    def _(j):
        dis_v[j] = zeros16

    pltpu.sync_copy(dis_v, sh_deg.at[pl.ds(s * NGRP, NGRP)])
    plsc.subcore_barrier()

    @pl.loop(0, NP // 16 // MERGE)
    def _(m):
        for k in range(MERGE // 16):
            idx_v[pl.ds(k * 16, 16)] = m * MERGE + k * 16 + iota16
        pltpu.sync_copy(degp_v.at[pl.ds(m * MERGE, MERGE)],
                        sh_deg.at[idx_v], add=True)

    plsc.subcore_barrier()

    pltpu.sync_copy(sh_deg.at[pl.ds(s * NGRP, NGRP)], dis_v)

    @pl.loop(0, NGRP)
    def _(j):
        dis_v[j] = _rsqrt16(dis_v[j])

    @pl.loop(0, ROW_BLK)
    def _(r):
        for f in range(DH // 16):
            xb_v[0, r, pl.ds(f * 16, 16)] = zeros16

    zouts = [pltpu.async_copy(
        xb_v.at[0], sh_out.at[pl.ds(nb + b * ROW_BLK, ROW_BLK)], sem_o.at[0])
        for b in range(BLKS_PER_TILE)]
    for dsc in zouts:
        dsc.wait()

    for t in range(BLKS_PER_TILE // 2):
        b0, b1 = 2 * t, 2 * t + 1
        din0 = pltpu.async_copy(
            xh.at[c, pl.ds(nb + b0 * ROW_BLK, ROW_BLK)], xb_v.at[0],
            sem_i.at[0])
        din1 = pltpu.async_copy(
            xh.at[c, pl.ds(nb + b1 * ROW_BLK, ROW_BLK)], xb_v.at[1],
            sem_i.at[1])
        din0.wait()
        _scale_blk(xb_v, 0, dis_v, b0 * ROW_BLK)
        dout0 = pltpu.async_copy(
            xb_v.at[0], y_h.at[c, pl.ds(nb + b0 * ROW_BLK, ROW_BLK)],
            sem_o.at[0])
        din1.wait()
        _scale_blk(xb_v, 1, dis_v, b1 * ROW_BLK)
        dout1 = pltpu.async_copy(
            xb_v.at[1], y_h.at[c, pl.ds(nb + b1 * ROW_BLK, ROW_BLK)],
            sem_o.at[1])
        dout0.wait()
        dout1.wait()

    plsc.subcore_barrier()

    def _fire_gather(t, j):
        return pltpu.async_copy(
            y_h.at[c].at[row2_v.at[t * NBUF + j]], bufs_v.at[j], sem_g.at[j])

    def _fire_scatter(t, j):
        return pltpu.async_copy(
            bufs_v.at[j], sh_out.at[col_v.at[t * NBUF + j]],
            sem_s.at[j], add=True)

    g0 = [_fire_gather(0, j) for j in range(NBUF)]
    for j in range(NBUF):
        g0[j].wait()
        _fire_scatter(0, j)

    @pl.loop(1, N_BATCHES)
    def _(t):
        gathers = []
        for j in range(NBUF):
            pltpu.make_async_copy(
                bufs_v.at[j],
                sh_out.at[col_v.at[(t - 1) * NBUF + j]],
                sem_s.at[j]).wait()
            gathers.append(_fire_gather(t, j))
        for j in range(NBUF):
            gathers[j].wait()
            _fire_scatter(t, j)

    for j in range(NBUF):
        pltpu.make_async_copy(
            bufs_v.at[j],
            sh_out.at[col_v.at[(N_BATCHES - 1) * NBUF + j]],
            sem_s.at[j]).wait()

    plsc.subcore_barrier()

    for t in range(BLKS_PER_TILE // 2):
        b0, b1 = 2 * t, 2 * t + 1
        din0 = pltpu.async_copy(
            sh_out.at[pl.ds(nb + b0 * ROW_BLK, ROW_BLK)], xb_v.at[0],
            sem_i.at[0])
        din1 = pltpu.async_copy(
            sh_out.at[pl.ds(nb + b1 * ROW_BLK, ROW_BLK)], xb_v.at[1],
            sem_i.at[1])
        din0.wait()
        _scale_blk(xb_v, 0, dis_v, b0 * ROW_BLK)
        dout0 = pltpu.async_copy(
            xb_v.at[0], out_h.at[c, pl.ds(nb + b0 * ROW_BLK, ROW_BLK)],
            sem_o.at[0])
        din1.wait()
        _scale_blk(xb_v, 1, dis_v, b1 * ROW_BLK)
        dout1 = pltpu.async_copy(
            xb_v.at[1], out_h.at[c, pl.ds(nb + b1 * ROW_BLK, ROW_BLK)],
            sem_o.at[1])
        dout0.wait()
        dout1.wait()


@jax.jit
def kernel(x, edge_index):
    n, d = x.shape
    assert n == N_NODES and d == D and edge_index.shape == (2, N_EDGES)

    x_pad = jnp.zeros((NP, d), x.dtype).at[:n].set(x)
    xh = jnp.stack([x_pad[:, :DH], x_pad[:, DH:]])
    row2 = edge_index[0].reshape(N_TILES, CHUNKS_PER_TILE, CHUNK)
    col2 = edge_index[1].reshape(N_TILES, CHUNKS_PER_TILE, CHUNK)

    mesh = plsc.VectorSubcoreMesh(core_axis_name="c", subcore_axis_name="s")
    run = functools.partial(
        pl.kernel,
        out_type=(jax.ShapeDtypeStruct((2, NP, DH), jnp.float32),
                  jax.ShapeDtypeStruct((2, NP, DH), jnp.float32)),
        mesh=mesh,
        compiler_params=pltpu.CompilerParams(
            needs_layout_passes=False, use_tc_tiling_on_sc=False),
        scratch_types=[
            pltpu.VMEM((CHUNKS_PER_TILE, CHUNK), jnp.int32),
            pltpu.VMEM((CHUNKS_PER_TILE, CHUNK), jnp.int32),
            pltpu.VMEM((NP // 16, 16), jnp.float32),
            pltpu.VMEM((NGRP, 16), jnp.float32),
            pltpu.VMEM((MERGE,), jnp.int32),
            pltpu.VMEM((NBUF, CHUNK, DH), jnp.float32),
            pltpu.VMEM((2, ROW_BLK, DH), jnp.float32),
            pltpu.SemaphoreType.DMA((NBUF,)),
            pltpu.SemaphoreType.DMA((NBUF,)),
            pltpu.SemaphoreType.DMA((2,)),
            pltpu.SemaphoreType.DMA((2,)),
            pltpu.VMEM_SHARED((NP // 16, 16), jnp.float32),
            pltpu.VMEM_SHARED((NP, DH), jnp.float32),
        ],
    )(_gcn_body)

    out2, _ = run(xh, row2, col2)
    return jnp.concatenate([out2[0], out2[1]], axis=1)[:n]


# speedup vs baseline: 1.0624x; 1.0624x over previous
import functools

import jax
import jax.numpy as jnp
from jax import lax
from jax.experimental import pallas as pl
from jax.experimental.pallas import tpu as pltpu
from jax.experimental.pallas import tpu_sc as plsc

N_NODES = 10000
N_EDGES = 320000
D = 128

NP = 10240
DH = D // 2
N_TILES = 16
NODES_PER_TILE = NP // N_TILES
EDGES_PER_TILE = N_EDGES // N_TILES
CHUNK = 80
CHUNKS_PER_TILE = EDGES_PER_TILE // CHUNK
NBUF = 5
N_BATCHES = CHUNKS_PER_TILE // NBUF
ROW_BLK = 64
BLKS_PER_TILE = NODES_PER_TILE // ROW_BLK
NGRP = NODES_PER_TILE // 16
MERGE = 128


def _rsqrt16(d):
    i = plsc.bitcast(d, jnp.int32)
    i = jnp.int32(0x5F3759DF) - (i >> 1)
    y = plsc.bitcast(i, jnp.float32)
    for _ in range(3):
        y = y * (1.5 - 0.5 * d * y * y)
    return jnp.where(d > 0.5, y, 0.0)


def _scale_blk(xb_v, p, dis_v, base):
    @pl.loop(0, ROW_BLK // 16)
    def _(j):
        dv = dis_v[base // 16 + j]
        for rr in range(16):
            dsc = dv[rr]
            for f in range(DH // 16):
                xb_v[p, j * 16 + rr, pl.ds(f * 16, 16)] = (
                    xb_v[p, j * 16 + rr, pl.ds(f * 16, 16)] * dsc)


def _gcn_body(xh, row2_h, col_h, out_h, y_h,
              row2_v, col_v, degp_v, dis_v, idx_v, bufs_v, xb_v,
              sem_g, sem_s, sem_i, sem_o,
              sh_deg, sh_out):
    c = lax.axis_index("c")
    s = lax.axis_index("s")
    nb = s * NODES_PER_TILE

    zeros16 = jnp.zeros((16,), jnp.float32)
    ones16 = jnp.ones((16,), jnp.float32)
    iota16 = lax.iota(jnp.int32, 16)

    d_row = pltpu.async_copy(row2_h.at[s], row2_v, sem_i.at[0])
    d_col = pltpu.async_copy(col_h.at[s], col_v, sem_i.at[1])
    d_row.wait()

    @pl.loop(0, NP // 16)
    def _(i):
        degp_v[i] = zeros16

    @pl.loop(0, CHUNKS_PER_TILE)
    def _(g):
        for k in range(CHUNK // 16):
            idx = row2_v[g, pl.ds(k * 16, 16)]
            plsc.addupdate_scatter(degp_v, [idx >> 4, idx & 15], ones16)

    @pl.loop(0, NGRP)
    def _(j):
        dis_v[j] = zeros16

    pltpu.sync_copy(dis_v, sh_deg.at[pl.ds(s * NGRP, NGRP)])
    plsc.subcore_barrier()

    @pl.loop(0, NP // 16 // MERGE)
    def _(m):
        for k in range(MERGE // 16):
            idx_v[pl.ds(k * 16, 16)] = m * MERGE + k * 16 + iota16
        pltpu.sync_copy(degp_v.at[pl.ds(m * MERGE, MERGE)],
                        sh_deg.at[idx_v], add=True)

    plsc.subcore_barrier()

    pltpu.sync_copy(sh_deg.at[pl.ds(s * NGRP, NGRP)], dis_v)

    @pl.loop(0, NGRP)
    def _(j):
        dis_v[j] = _rsqrt16(dis_v[j])

    @pl.loop(0, ROW_BLK)
    def _(r):
        for f in range(DH // 16):
            xb_v[0, r, pl.ds(f * 16, 16)] = zeros16

    zouts = [pltpu.async_copy(
        xb_v.at[0], sh_out.at[pl.ds(nb + b * ROW_BLK, ROW_BLK)], sem_o.at[0])
        for b in range(BLKS_PER_TILE)]
    for dsc in zouts:
        dsc.wait()

    for t in range(BLKS_PER_TILE // 2):
        b0, b1 = 2 * t, 2 * t + 1
        din0 = pltpu.async_copy(
            xh.at[c, pl.ds(nb + b0 * ROW_BLK, ROW_BLK)], xb_v.at[0],
            sem_i.at[0])
        din1 = pltpu.async_copy(
            xh.at[c, pl.ds(nb + b1 * ROW_BLK, ROW_BLK)], xb_v.at[1],
            sem_i.at[1])
        din0.wait()
        _scale_blk(xb_v, 0, dis_v, b0 * ROW_BLK)
        dout0 = pltpu.async_copy(
            xb_v.at[0], y_h.at[c, pl.ds(nb + b0 * ROW_BLK, ROW_BLK)],
            sem_o.at[0])
        din1.wait()
        _scale_blk(xb_v, 1, dis_v, b1 * ROW_BLK)
        dout1 = pltpu.async_copy(
            xb_v.at[1], y_h.at[c, pl.ds(nb + b1 * ROW_BLK, ROW_BLK)],
            sem_o.at[1])
        dout0.wait()
        dout1.wait()

    plsc.subcore_barrier()

    def _fire_gather(t, j):
        return pltpu.async_copy(
            y_h.at[c].at[row2_v.at[t * NBUF + j]], bufs_v.at[j], sem_g.at[j])

    def _fire_scatter(t, j):
        return pltpu.async_copy(
            bufs_v.at[j], sh_out.at[col_v.at[t * NBUF + j]],
            sem_s.at[j], add=True)

    d_col.wait()
    g0 = [_fire_gather(0, j) for j in range(NBUF)]
    for j in range(NBUF):
        g0[j].wait()
        _fire_scatter(0, j)

    @pl.loop(1, N_BATCHES)
    def _(t):
        gathers = []
        for j in range(NBUF):
            pltpu.make_async_copy(
                bufs_v.at[j],
                sh_out.at[col_v.at[(t - 1) * NBUF + j]],
                sem_s.at[j]).wait()
            gathers.append(_fire_gather(t, j))
        for j in range(NBUF):
            gathers[j].wait()
            _fire_scatter(t, j)

    for j in range(NBUF):
        pltpu.make_async_copy(
            bufs_v.at[j],
            sh_out.at[col_v.at[(N_BATCHES - 1) * NBUF + j]],
            sem_s.at[j]).wait()

    plsc.subcore_barrier()

    for t in range(BLKS_PER_TILE // 2):
        b0, b1 = 2 * t, 2 * t + 1
        din0 = pltpu.async_copy(
            sh_out.at[pl.ds(nb + b0 * ROW_BLK, ROW_BLK)], xb_v.at[0],
            sem_i.at[0])
        din1 = pltpu.async_copy(
            sh_out.at[pl.ds(nb + b1 * ROW_BLK, ROW_BLK)], xb_v.at[1],
            sem_i.at[1])
        din0.wait()
        _scale_blk(xb_v, 0, dis_v, b0 * ROW_BLK)
        dout0 = pltpu.async_copy(
            xb_v.at[0], out_h.at[c, pl.ds(nb + b0 * ROW_BLK, ROW_BLK)],
            sem_o.at[0])
        din1.wait()
        _scale_blk(xb_v, 1, dis_v, b1 * ROW_BLK)
        dout1 = pltpu.async_copy(
            xb_v.at[1], out_h.at[c, pl.ds(nb + b1 * ROW_BLK, ROW_BLK)],
            sem_o.at[1])
        dout0.wait()
        dout1.wait()


@jax.jit
def kernel(x, edge_index):
    n, d = x.shape
    assert n == N_NODES and d == D and edge_index.shape == (2, N_EDGES)

    x_pad = jnp.zeros((NP, d), x.dtype).at[:n].set(x)
    xh = jnp.stack([x_pad[:, :DH], x_pad[:, DH:]])
    row2 = edge_index[0].reshape(N_TILES, CHUNKS_PER_TILE, CHUNK)
    col2 = edge_index[1].reshape(N_TILES, CHUNKS_PER_TILE, CHUNK)

    mesh = plsc.VectorSubcoreMesh(core_axis_name="c", subcore_axis_name="s")
    run = functools.partial(
        pl.kernel,
        out_type=(jax.ShapeDtypeStruct((2, NP, DH), jnp.float32),
                  jax.ShapeDtypeStruct((2, NP, DH), jnp.float32)),
        mesh=mesh,
        compiler_params=pltpu.CompilerParams(
            needs_layout_passes=False, use_tc_tiling_on_sc=False),
        scratch_types=[
            pltpu.VMEM((CHUNKS_PER_TILE, CHUNK), jnp.int32),
            pltpu.VMEM((CHUNKS_PER_TILE, CHUNK), jnp.int32),
            pltpu.VMEM((NP // 16, 16), jnp.float32),
            pltpu.VMEM((NGRP, 16), jnp.float32),
            pltpu.VMEM((MERGE,), jnp.int32),
            pltpu.VMEM((NBUF, CHUNK, DH), jnp.float32),
            pltpu.VMEM((2, ROW_BLK, DH), jnp.float32),
            pltpu.SemaphoreType.DMA((NBUF,)),
            pltpu.SemaphoreType.DMA((NBUF,)),
            pltpu.SemaphoreType.DMA((2,)),
            pltpu.SemaphoreType.DMA((2,)),
            pltpu.VMEM_SHARED((NP // 16, 16), jnp.float32),
            pltpu.VMEM_SHARED((NP, DH), jnp.float32),
        ],
    )(_gcn_body)

    out2, _ = run(xh, row2, col2)
    return jnp.concatenate([out2[0], out2[1]], axis=1)[:n]
